# baseline (speedup 1.0000x reference)
import functools

import jax
import jax.numpy as jnp
from jax import lax
from jax.experimental import pallas as pl
from jax.experimental.pallas import tpu as pltpu
from jax.experimental.pallas import tpu_sc as plsc

N = 10000
E = 160000
IN = 128
HID = 8

NC = 2
NS = 16
NW = NC * NS
CHUNK = 128
E_PAD = 163840
EPW = E_PAD // NW
NCHUNK = EPW // CHUNK
N_PAD = 10240
AGG = N_PAD * HID


def _node_pre_body(h_ref, a_ref, root_ref, bias_ref, t_ref, r_ref):
    h = h_ref[...]
    t_ref[...] = jnp.dot(h, a_ref[...])
    r_ref[...] = jnp.dot(h, root_ref[...]) + bias_ref[...]


def _node_pre(h, a, root, bias, bn):
    rows = h.shape[0]
    d = h.shape[1]
    grid = (rows // bn,)
    return pl.pallas_call(
        _node_pre_body,
        grid=grid,
        in_specs=[
            pl.BlockSpec((bn, d), lambda i: (i, 0)),
            pl.BlockSpec((d, 80), lambda i: (0, 0)),
            pl.BlockSpec((d, HID), lambda i: (0, 0)),
            pl.BlockSpec((1, HID), lambda i: (0, 0)),
        ],
        out_specs=[
            pl.BlockSpec((bn, 80), lambda i: (i, 0)),
            pl.BlockSpec((bn, HID), lambda i: (i, 0)),
        ],
        out_shape=[
            jax.ShapeDtypeStruct((rows, 80), jnp.float32),
            jax.ShapeDtypeStruct((rows, HID), jnp.float32),
        ],
    )(h, a, root, bias)


BE = 2048


def _edge_prep_body(ei_ref, ea_ref, src_ref, dst_ref, ea_ref_out):
    i = pl.program_id(0)
    gid = i * BE + jax.lax.broadcasted_iota(jnp.int32, (BE,), 0)
    valid = gid < E
    src_ref[...] = jnp.where(valid, ei_ref[0, :], N)
    dst_ref[...] = jnp.where(valid, ei_ref[1, :], 0)
    ea = ea_ref[...]
    ea_ref_out[pl.ds(0, BE)] = ea[:, 0]
    ea_ref_out[pl.ds(BE, BE)] = ea[:, 1]


def _edge_prep(edge_index, ea2):
    grid = (E_PAD // BE,)
    nin = (E - 1) // BE
    return pl.pallas_call(
        _edge_prep_body,
        grid=grid,
        in_specs=[
            pl.BlockSpec((2, BE), lambda i: (0, jnp.minimum(i, nin))),
            pl.BlockSpec((BE, 2), lambda i: (jnp.minimum(i, nin), 0)),
        ],
        out_specs=[
            pl.BlockSpec((BE,), lambda i: (i,)),
            pl.BlockSpec((BE,), lambda i: (i,)),
            pl.BlockSpec((2 * BE,), lambda i: (i,)),
        ],
        out_shape=[
            jax.ShapeDtypeStruct((E_PAD,), jnp.int32),
            jax.ShapeDtypeStruct((E_PAD,), jnp.int32),
            jax.ShapeDtypeStruct((2 * E_PAD,), jnp.float32),
        ],
    )(edge_index, ea2)


def _combine_body(agg_ref, r_ref, h_ref):
    h_ref[...] = jnp.maximum(jnp.sum(agg_ref[...], axis=0) + r_ref[...], 0.0)


def _combine(agg, r_pad):
    bn = 1024
    grid = (N_PAD // bn,)
    return pl.pallas_call(
        _combine_body,
        grid=grid,
        in_specs=[
            pl.BlockSpec((NC, bn, HID), lambda i: (0, i, 0)),
            pl.BlockSpec((bn, HID), lambda i: (i, 0)),
        ],
        out_specs=pl.BlockSpec((bn, HID), lambda i: (i, 0)),
        out_shape=jax.ShapeDtypeStruct((N_PAD, HID), jnp.float32),
    )(agg, r_pad)


_mesh = plsc.VectorSubcoreMesh(core_axis_name="c", subcore_axis_name="s",
                               num_cores=NC, num_subcores=NS)


@functools.partial(
    pl.kernel,
    out_type=jax.ShapeDtypeStruct((NC, N_PAD, HID), jnp.float32),
    mesh=_mesh,
    compiler_params=pltpu.CompilerParams(
        needs_layout_passes=False, use_tc_tiling_on_sc=False),
    scratch_types=[
        [pltpu.VMEM((CHUNK,), jnp.int32) for _ in range(4)],
        [pltpu.VMEM((CHUNK,), jnp.int32) for _ in range(4)],
        [pltpu.VMEM((CHUNK,), jnp.float32) for _ in range(4)],
        [pltpu.VMEM((CHUNK,), jnp.float32) for _ in range(4)],
        [pltpu.VMEM((CHUNK, 80), jnp.float32) for _ in range(4)],
        [pltpu.VMEM((CHUNK, HID), jnp.float32) for _ in range(2)],
        [pltpu.VMEM((CHUNK,), jnp.int32) for _ in range(2)],
        pltpu.VMEM((24,), jnp.float32),
        pltpu.VMEM_SHARED((N_PAD, HID), jnp.float32),
        [pltpu.SemaphoreType.DMA for _ in range(4)],
        [pltpu.SemaphoreType.DMA for _ in range(4)],
        [pltpu.SemaphoreType.DMA for _ in range(2)],
    ],
)
def _edge_pass(t_hbm, ea_hbm, w_hbm, src_hbm, dst_hbm, zero_hbm,
               out_hbm, src_v, dst_v, ea_v, ea1_v, rows_v, msg_v, dsts_v,
               w_v, agg_sh, sem_i, sem_g, sem_s):
    c = lax.axis_index("c")
    s = lax.axis_index("s")
    base = jnp.where(c == 0, s * (48 * CHUNK),
                     16 * (48 * CHUNK) + s * (32 * CHUNK))
    nch = jnp.where(c == 0, 48, 32)
    RPT = N_PAD // NS

    pltpu.sync_copy(zero_hbm.at[pl.ds(s * RPT, RPT)],
                    agg_sh.at[pl.ds(s * RPT, RPT)])
    pltpu.sync_copy(w_hbm, w_v)
    wlo = w_v[pl.ds(0, 16)]
    whi = w_v[pl.ds(8, 16)]
    w1a = [wlo[h] for h in range(HID)]
    w1b = [wlo[HID + h] for h in range(HID)]
    b1 = [whi[HID + h] for h in range(HID)]
    plsc.subcore_barrier()

    def _ea_off(ci):
        off = base + ci * CHUNK
        return pl.multiple_of((off // 2048) * 4096 + (off % 2048), 128)

    def issue_in(r, ci):
        off = base + ci * CHUNK
        pltpu.async_copy(src_hbm.at[pl.ds(off, CHUNK)], src_v[r], sem_i[r])
        pltpu.async_copy(dst_hbm.at[pl.ds(off, CHUNK)], dst_v[r], sem_i[r])
        eo = _ea_off(ci)
        pltpu.async_copy(ea_hbm.at[pl.ds(eo, CHUNK)], ea_v[r], sem_i[r])
        pltpu.async_copy(ea_hbm.at[pl.ds(eo + 2048, CHUNK)], ea1_v[r],
                         sem_i[r])

    def wait_in(r, ci):
        off = base + ci * CHUNK
        pltpu.make_async_copy(src_hbm.at[pl.ds(off, CHUNK)], src_v[r],
                              sem_i[r]).wait()
        pltpu.make_async_copy(dst_hbm.at[pl.ds(off, CHUNK)], dst_v[r],
                              sem_i[r]).wait()
        eo = _ea_off(ci)
        pltpu.make_async_copy(ea_hbm.at[pl.ds(eo, CHUNK)], ea_v[r],
                              sem_i[r]).wait()
        pltpu.make_async_copy(ea_hbm.at[pl.ds(eo + 2048, CHUNK)], ea1_v[r],
                              sem_i[r]).wait()

    def issue_gather(r):
        pltpu.async_copy(t_hbm.at[src_v[r]], rows_v[r], sem_g[r])

    def wait_gather(r):
        pltpu.make_async_copy(t_hbm.at[src_v[r]], rows_v[r], sem_g[r]).wait()

    def issue_scatter(b):
        pltpu.async_copy(msg_v[b], agg_sh.at[dsts_v[b]], sem_s[b], add=True)

    def wait_scatter(b):
        pltpu.make_async_copy(msg_v[b], agg_sh.at[dsts_v[b]],
                              sem_s[b]).wait()

    def compute(r, b):
        lanes = lax.iota(jnp.int32, 16)

        def group(g, carry):
            g16 = g * 16
            rows = lanes + g16
            dsts_v[b][pl.ds(g16, 16)] = dst_v[r][pl.ds(g16, 16)]
            ea0 = plsc.load_gather(ea_v[r], [rows])
            ea1 = plsc.load_gather(ea1_v[r], [rows])
            hhg = [jnp.maximum(ea0 * w1a[h] + ea1 * w1b[h] + b1[h], 0.0)
                   for h in range(HID)]
            for o in range(HID):
                acc = plsc.load_gather(
                    rows_v[r], [rows, jnp.full((16,), 64 + o, jnp.int32)])
                for h in range(HID):
                    t = plsc.load_gather(
                        rows_v[r], [rows, jnp.full((16,), h * 8 + o,
                                                   jnp.int32)])
                    acc = acc + hhg[h] * t
                plsc.store_scatter(msg_v[b],
                                   [rows, jnp.full((16,), o, jnp.int32)], acc)
            return carry

        lax.fori_loop(0, CHUNK // 16, group, 0)

    for r in range(4):
        issue_in(r, r)
    for r in range(3):
        wait_in(r, r)
        issue_gather(r)

    def outer(j, carry):
        for r in range(4):
            ci = j * 4 + r
            b = r % 2
            wait_gather(r)

            @pl.when(ci >= 2)
            def _():
                wait_scatter(b)

            @pl.when(ci + 3 < nch)
            def _():
                wait_in((r + 3) % 4, ci + 3)
                issue_gather((r + 3) % 4)

            compute(r, b)
            issue_scatter(b)

            @pl.when(ci + 4 < nch)
            def _():
                issue_in(r, ci + 4)
        return carry

    lax.fori_loop(0, nch // 4, outer, 0)
    wait_scatter(0)
    wait_scatter(1)
    plsc.subcore_barrier()
    pltpu.sync_copy(agg_sh.at[pl.ds(s * RPT, RPT)],
                    out_hbm.at[c, pl.ds(s * RPT, RPT)])


def _table_weights(w2, b2, in_ch):
    a = w2.reshape(HID, in_ch, HID).transpose(1, 0, 2).reshape(in_ch, 64)
    return jnp.concatenate(
        [a, b2.reshape(in_ch, HID), jnp.zeros((in_ch, 8), jnp.float32)],
        axis=1)


def _pad_nodes(t, r):
    pad = N_PAD - N
    return (jnp.pad(t, ((0, pad), (0, 0))), jnp.pad(r, ((0, pad), (0, 0))))


def kernel(x, edge_index, edge_attr, l1_w1, l1_b1, l1_w2, l1_b2, l1_root,
           l1_bias, l2_w1, l2_b1, l2_w2, l2_b2, l2_root, l2_bias):
    srcp, dstp, ea_flat = _edge_prep(edge_index.astype(jnp.int32),
                                     edge_attr.reshape(E, 2))

    a1 = _table_weights(l1_w2, l1_b2, IN)
    a2 = _table_weights(l2_w2, l2_b2, HID)
    zero_n8 = jnp.zeros((N_PAD, HID), jnp.float32)

    wb1 = jnp.concatenate([l1_w1.reshape(2 * HID), l1_b1])
    wb2 = jnp.concatenate([l2_w1.reshape(2 * HID), l2_b1])

    t1, r1 = _node_pre(x, a1, l1_root, l1_bias.reshape(1, HID), 1000)
    t1p, r1p = _pad_nodes(t1, r1)
    agg1 = _edge_pass(t1p, ea_flat, wb1, srcp, dstp, zero_n8)
    h1 = _combine(agg1, r1p)

    t2, r2 = _node_pre(h1, a2, l2_root, l2_bias.reshape(1, HID), 1024)
    agg2 = _edge_pass(t2, ea_flat, wb2, srcp, dstp, zero_n8)
    h2 = _combine(agg2, r2)
    return h2[:N]


# speedup vs baseline: 1.0258x; 1.0258x over previous
import functools

import jax
import jax.numpy as jnp
from jax import lax
from jax.experimental import pallas as pl
from jax.experimental.pallas import tpu as pltpu
from jax.experimental.pallas import tpu_sc as plsc

N = 10000
E = 160000
IN = 128
HID = 8

NC = 2
NS = 16
NW = NC * NS
CHUNK = 128
E_PAD = 163840
EPW = E_PAD // NW
NCHUNK = EPW // CHUNK
N_PAD = 10240
AGG = N_PAD * HID


def _node_pre_body(h_ref, a_ref, root_ref, bias_ref, t_ref, r_ref):
    h = h_ref[...]
    t_ref[...] = jnp.dot(h, a_ref[...])
    r_ref[...] = jnp.dot(h, root_ref[...]) + bias_ref[...]


def _node_pre(h, a, root, bias, bn):
    rows = h.shape[0]
    d = h.shape[1]
    grid = (rows // bn,)
    return pl.pallas_call(
        _node_pre_body,
        grid=grid,
        in_specs=[
            pl.BlockSpec((bn, d), lambda i: (i, 0)),
            pl.BlockSpec((d, 80), lambda i: (0, 0)),
            pl.BlockSpec((d, HID), lambda i: (0, 0)),
            pl.BlockSpec((1, HID), lambda i: (0, 0)),
        ],
        out_specs=[
            pl.BlockSpec((bn, 80), lambda i: (i, 0)),
            pl.BlockSpec((bn, HID), lambda i: (i, 0)),
        ],
        out_shape=[
            jax.ShapeDtypeStruct((rows, 80), jnp.float32),
            jax.ShapeDtypeStruct((rows, HID), jnp.float32),
        ],
    )(h, a, root, bias)


BE = 2048


def _edge_prep_body(ei_ref, src_ref, dst_ref):
    i = pl.program_id(0)
    gid = i * BE + jax.lax.broadcasted_iota(jnp.int32, (BE,), 0)
    valid = gid < E
    src_ref[...] = jnp.where(valid, ei_ref[0, :], N)
    dst_ref[...] = jnp.where(valid, ei_ref[1, :], 0)


def _edge_prep(edge_index):
    grid = (E_PAD // BE,)
    nin = (E - 1) // BE
    return pl.pallas_call(
        _edge_prep_body,
        grid=grid,
        in_specs=[
            pl.BlockSpec((2, BE), lambda i: (0, jnp.minimum(i, nin))),
        ],
        out_specs=[
            pl.BlockSpec((BE,), lambda i: (i,)),
            pl.BlockSpec((BE,), lambda i: (i,)),
        ],
        out_shape=[
            jax.ShapeDtypeStruct((E_PAD,), jnp.int32),
            jax.ShapeDtypeStruct((E_PAD,), jnp.int32),
        ],
    )(edge_index)


def _combine_body(agg_ref, r_ref, h_ref):
    h_ref[...] = jnp.maximum(jnp.sum(agg_ref[...], axis=0) + r_ref[...], 0.0)


def _combine(agg, r_pad):
    bn = 1024
    grid = (N_PAD // bn,)
    return pl.pallas_call(
        _combine_body,
        grid=grid,
        in_specs=[
            pl.BlockSpec((NC, bn, HID), lambda i: (0, i, 0)),
            pl.BlockSpec((bn, HID), lambda i: (i, 0)),
        ],
        out_specs=pl.BlockSpec((bn, HID), lambda i: (i, 0)),
        out_shape=jax.ShapeDtypeStruct((N_PAD, HID), jnp.float32),
    )(agg, r_pad)


_mesh = plsc.VectorSubcoreMesh(core_axis_name="c", subcore_axis_name="s",
                               num_cores=NC, num_subcores=NS)


@functools.partial(
    pl.kernel,
    out_type=jax.ShapeDtypeStruct((NC, N_PAD, HID), jnp.float32),
    mesh=_mesh,
    compiler_params=pltpu.CompilerParams(
        needs_layout_passes=False, use_tc_tiling_on_sc=False),
    scratch_types=[
        [pltpu.VMEM((CHUNK,), jnp.int32) for _ in range(4)],
        [pltpu.VMEM((CHUNK,), jnp.int32) for _ in range(4)],
        [pltpu.VMEM((2 * CHUNK,), jnp.float32) for _ in range(4)],
        [pltpu.VMEM((CHUNK, 80), jnp.float32) for _ in range(4)],
        [pltpu.VMEM((CHUNK, HID), jnp.float32) for _ in range(2)],
        [pltpu.VMEM((CHUNK,), jnp.int32) for _ in range(2)],
        pltpu.VMEM((24,), jnp.float32),
        pltpu.VMEM_SHARED((N_PAD, HID), jnp.float32),
        [pltpu.SemaphoreType.DMA for _ in range(4)],
        [pltpu.SemaphoreType.DMA for _ in range(4)],
        [pltpu.SemaphoreType.DMA for _ in range(2)],
    ],
)
def _edge_pass(t_hbm, ea_hbm, w_hbm, src_hbm, dst_hbm, zero_hbm,
               out_hbm, src_v, dst_v, ea_v, rows_v, msg_v, dsts_v,
               w_v, agg_sh, sem_i, sem_g, sem_s):
    c = lax.axis_index("c")
    s = lax.axis_index("s")
    base = jnp.where(c == 0, s * (48 * CHUNK),
                     16 * (48 * CHUNK) + s * (32 * CHUNK))
    nch = jnp.where(c == 0, 48, 32)
    RPT = N_PAD // NS

    pltpu.sync_copy(zero_hbm.at[pl.ds(s * RPT, RPT)],
                    agg_sh.at[pl.ds(s * RPT, RPT)])
    pltpu.sync_copy(w_hbm, w_v)
    wlo = w_v[pl.ds(0, 16)]
    whi = w_v[pl.ds(8, 16)]
    w1a = [wlo[h] for h in range(HID)]
    w1b = [wlo[HID + h] for h in range(HID)]
    b1 = [whi[HID + h] for h in range(HID)]
    plsc.subcore_barrier()

    def _ea_off(ci):
        off = base + ci * CHUNK
        return jnp.minimum(off, E - CHUNK) * 2

    def issue_in(r, ci):
        off = base + ci * CHUNK
        pltpu.async_copy(src_hbm.at[pl.ds(off, CHUNK)], src_v[r], sem_i[r])
        pltpu.async_copy(dst_hbm.at[pl.ds(off, CHUNK)], dst_v[r], sem_i[r])
        pltpu.async_copy(ea_hbm.at[pl.ds(_ea_off(ci), 2 * CHUNK)], ea_v[r],
                         sem_i[r])

    def wait_in(r, ci):
        off = base + ci * CHUNK
        pltpu.make_async_copy(src_hbm.at[pl.ds(off, CHUNK)], src_v[r],
                              sem_i[r]).wait()
        pltpu.make_async_copy(dst_hbm.at[pl.ds(off, CHUNK)], dst_v[r],
                              sem_i[r]).wait()
        pltpu.make_async_copy(ea_hbm.at[pl.ds(_ea_off(ci), 2 * CHUNK)],
                              ea_v[r], sem_i[r]).wait()

    def issue_gather(r):
        pltpu.async_copy(t_hbm.at[src_v[r]], rows_v[r], sem_g[r])

    def wait_gather(r):
        pltpu.make_async_copy(t_hbm.at[src_v[r]], rows_v[r], sem_g[r]).wait()

    def issue_scatter(b):
        pltpu.async_copy(msg_v[b], agg_sh.at[dsts_v[b]], sem_s[b], add=True)

    def wait_scatter(b):
        pltpu.make_async_copy(msg_v[b], agg_sh.at[dsts_v[b]],
                              sem_s[b]).wait()

    def compute(r, b):
        lanes = lax.iota(jnp.int32, 16)

        def group(g, carry):
            g16 = g * 16
            rows = lanes + g16
            dsts_v[b][pl.ds(g16, 16)] = dst_v[r][pl.ds(g16, 16)]
            ea0 = plsc.load_gather(ea_v[r], [rows * 2])
            ea1 = plsc.load_gather(ea_v[r], [rows * 2 + 1])
            hhg = [jnp.maximum(ea0 * w1a[h] + ea1 * w1b[h] + b1[h], 0.0)
                   for h in range(HID)]
            for o in range(HID):
                acc = plsc.load_gather(
                    rows_v[r], [rows, jnp.full((16,), 64 + o, jnp.int32)])
                for h in range(HID):
                    t = plsc.load_gather(
                        rows_v[r], [rows, jnp.full((16,), h * 8 + o,
                                                   jnp.int32)])
                    acc = acc + hhg[h] * t
                plsc.store_scatter(msg_v[b],
                                   [rows, jnp.full((16,), o, jnp.int32)], acc)
            return carry

        lax.fori_loop(0, CHUNK // 16, group, 0)

    for r in range(4):
        issue_in(r, r)
    for r in range(2):
        wait_in(r, r)
        issue_gather(r)

    def outer(j, carry):
        for r in range(4):
            ci = j * 4 + r
            b = r % 2
            wait_gather(r)

            @pl.when(ci >= 2)
            def _():
                wait_scatter(b)

            @pl.when(ci + 2 < nch)
            def _():
                wait_in((r + 2) % 4, ci + 2)
                issue_gather((r + 2) % 4)

            compute(r, b)
            issue_scatter(b)

            @pl.when(ci + 4 < nch)
            def _():
                issue_in(r, ci + 4)
        return carry

    lax.fori_loop(0, nch // 4, outer, 0)
    wait_scatter(0)
    wait_scatter(1)
    plsc.subcore_barrier()
    pltpu.sync_copy(agg_sh.at[pl.ds(s * RPT, RPT)],
                    out_hbm.at[c, pl.ds(s * RPT, RPT)])


def _table_weights(w2, b2, in_ch):
    a = w2.reshape(HID, in_ch, HID).transpose(1, 0, 2).reshape(in_ch, 64)
    return jnp.concatenate(
        [a, b2.reshape(in_ch, HID), jnp.zeros((in_ch, 8), jnp.float32)],
        axis=1)


def _pad_nodes(t, r):
    pad = N_PAD - N
    return (jnp.pad(t, ((0, pad), (0, 0))), jnp.pad(r, ((0, pad), (0, 0))))


def kernel(x, edge_index, edge_attr, l1_w1, l1_b1, l1_w2, l1_b2, l1_root,
           l1_bias, l2_w1, l2_b1, l2_w2, l2_b2, l2_root, l2_bias):
    srcp, dstp = _edge_prep(edge_index.astype(jnp.int32))
    ea_flat = edge_attr.reshape(2 * E)

    a1 = _table_weights(l1_w2, l1_b2, IN)
    a2 = _table_weights(l2_w2, l2_b2, HID)
    zero_n8 = jnp.zeros((N_PAD, HID), jnp.float32)

    wb1 = jnp.concatenate([l1_w1.reshape(2 * HID), l1_b1])
    wb2 = jnp.concatenate([l2_w1.reshape(2 * HID), l2_b1])

    t1, r1 = _node_pre(x, a1, l1_root, l1_bias.reshape(1, HID), 1000)
    t1p, r1p = _pad_nodes(t1, r1)
    agg1 = _edge_pass(t1p, ea_flat, wb1, srcp, dstp, zero_n8)
    h1 = _combine(agg1, r1p)

    t2, r2 = _node_pre(h1, a2, l2_root, l2_bias.reshape(1, HID), 1024)
    agg2 = _edge_pass(t2, ea_flat, wb2, srcp, dstp, zero_n8)
    h2 = _combine(agg2, r2)
    return h2[:N]
